# baseline (speedup 1.0000x reference)
import functools

import jax
import jax.numpy as jnp
from jax import lax
from jax.experimental import pallas as pl
from jax.experimental.pallas import tpu as pltpu
from jax.experimental.pallas import tpu_sc as plsc

NC = 2
NS = 16
NW = NC * NS
CH = 128


def _round_up(a, b):
    return (a + b - 1) // b * b


def _mesh():
    return plsc.VectorSubcoreMesh(
        core_axis_name="c", subcore_axis_name="s", num_cores=NC, num_subcores=NS
    )


def _make_stats_kernel(n_pad, d, chunks_a):
    npt = n_pad // NW
    nps = n_pad // NS

    @functools.partial(
        pl.kernel,
        out_type=[
            jax.ShapeDtypeStruct((n_pad,), jnp.float32),
            jax.ShapeDtypeStruct((n_pad, d), jnp.float32),
        ],
        mesh=_mesh(),
        scratch_types=[
            pltpu.VMEM_SHARED((n_pad,), jnp.float32),
            pltpu.VMEM((chunks_a, CH), jnp.int32),
            pltpu.VMEM((CH,), jnp.float32),
            pltpu.VMEM((nps,), jnp.float32),
            pltpu.VMEM((npt,), jnp.float32),
            pltpu.VMEM((npt,), jnp.float32),
            pltpu.VMEM((npt, d), jnp.float32),
            pltpu.SemaphoreType.DMA,
            pltpu.SemaphoreType.DMA,
        ],
    )
    def stats(x_hbm, ei_hbm, dis_out, xs_out,
              deg_sh, idx_v, ones_v, zz_v, deg_v, dis_v, xrow_v, sem, sem_x):
        c = lax.axis_index("c")
        s = lax.axis_index("s")
        w = c * NS + s

        xdesc = pltpu.async_copy(x_hbm.at[pl.ds(w * npt, npt), :], xrow_v, sem_x)

        def zloop(i, carry):
            zz_v[pl.ds(i * 16, 16)] = jnp.zeros((16,), jnp.float32)
            return carry

        lax.fori_loop(0, nps // 16, zloop, 0)
        pltpu.sync_copy(zz_v, deg_sh.at[pl.ds(s * nps, nps)])

        def oloop(i, carry):
            ones_v[pl.ds(i * 16, 16)] = jnp.ones((16,), jnp.float32)
            return carry

        lax.fori_loop(0, CH // 16, oloop, 0)
        pltpu.sync_copy(ei_hbm.at[0, pl.ds(s * chunks_a, chunks_a), :], idx_v)
        plsc.subcore_barrier()

        k = 160
        for g in range(0, chunks_a, k):
            descs = [
                pltpu.async_copy(ones_v, deg_sh.at[idx_v.at[j]], sem, add=True)
                for j in range(g, min(g + k, chunks_a))
            ]
            for dsc in descs:
                dsc.wait()
        plsc.subcore_barrier()

        pltpu.sync_copy(deg_sh.at[pl.ds(w * npt, npt)], deg_v)

        def dloop(i, carry):
            dvec = deg_v[pl.ds(i * 16, 16)]
            ib = lax.bitcast_convert_type(dvec, jnp.int32)
            y = lax.bitcast_convert_type(
                jnp.int32(0x5F3759DF) - (ib >> 1), jnp.float32
            )
            half = 0.5 * dvec
            y = y * (1.5 - half * y * y)
            y = y * (1.5 - half * y * y)
            y = y * (1.5 - half * y * y)
            dis_v[pl.ds(i * 16, 16)] = jnp.where(dvec > 0.5, y, 0.0)
            return carry

        lax.fori_loop(0, npt // 16, dloop, 0)
        pltpu.sync_copy(dis_v, dis_out.at[pl.ds(w * npt, npt)])

        xdesc.wait()

        def xloop(g, carry):
            dvec = dis_v[pl.ds(g * 16, 16)]
            for l in range(16):
                dv = dvec[l]
                r = g * 16 + l
                for k in range(d // 16):
                    xrow_v[r, pl.ds(k * 16, 16)] = (
                        xrow_v[r, pl.ds(k * 16, 16)] * dv
                    )
            return carry

        lax.fori_loop(0, npt // 16, xloop, 0)
        pltpu.sync_copy(xrow_v, xs_out.at[pl.ds(w * npt, npt), :])

    return stats


def _make_spmm_kernel(n_pad, d, chunks_c):
    nps = n_pad // NS

    nq = 10
    qc = chunks_c // nq

    @functools.partial(
        pl.kernel,
        out_type=jax.ShapeDtypeStruct((NC, n_pad, d), jnp.float32),
        mesh=_mesh(),
        scratch_types=[
            pltpu.VMEM_SHARED((n_pad, d), jnp.float32),
            pltpu.VMEM((qc, CH), jnp.int32),
            pltpu.VMEM((qc, CH), jnp.int32),
            pltpu.VMEM((qc, CH), jnp.int32),
            pltpu.VMEM((qc, CH), jnp.int32),
            pltpu.VMEM((CH, d), jnp.float32),
            pltpu.VMEM((CH, d), jnp.float32),
            pltpu.SemaphoreType.DMA,
            pltpu.SemaphoreType.DMA,
            pltpu.SemaphoreType.DMA,
            pltpu.SemaphoreType.DMA,
        ],
    )
    def spmm(xs_hbm, ei_hbm, tx1_out,
             tx1_sh, ridx0, cidx0, ridx1, cidx1, buf0, buf1,
             sem0, sem1, isem0, isem1):
        c = lax.axis_index("c")
        s = lax.axis_index("s")
        w = c * NS + s

        pairs = ((ridx0, cidx0, isem0), (ridx1, cidx1, isem1))

        def stage(q):
            r, ci, sm = pairs[q % 2]
            base = w * chunks_c + q * qc
            return (
                pltpu.async_copy(ei_hbm.at[0, pl.ds(base, qc), :], r, sm),
                pltpu.async_copy(ei_hbm.at[1, pl.ds(base, qc), :], ci, sm),
            )

        idescs = [None] * nq
        idescs[0] = stage(0)

        def zrow(r, carry):
            for k in range(d // 16):
                buf0[r, pl.ds(k * 16, 16)] = jnp.zeros((16,), jnp.float32)
            return carry

        lax.fori_loop(0, CH, zrow, 0)
        for i in range(nps // CH):
            pltpu.sync_copy(buf0, tx1_sh.at[pl.ds(s * nps + i * CH, CH), :])
        plsc.subcore_barrier()

        for dsc in idescs[0]:
            dsc.wait()
        if nq > 1:
            idescs[1] = stage(1)

        bufs = (buf0, buf1)
        sems = (sem0, sem1)
        g_descs = [None, None]
        g_descs[0] = pltpu.async_copy(xs_hbm.at[ridx0.at[0]], buf0, sem0)
        for j in range(chunks_c):
            q, jj = j // qc, j % qc
            cme = pairs[q % 2][1]
            cur, nxt = j % 2, (j + 1) % 2
            if j + 1 < chunks_c:
                q2 = (j + 1) // qc
                if q2 != q:
                    for dsc in idescs[q2]:
                        dsc.wait()
                rn = pairs[q2 % 2][0]
                g_descs[nxt] = pltpu.async_copy(
                    xs_hbm.at[rn.at[(j + 1) % qc]], bufs[nxt], sems[nxt]
                )
            g_descs[cur].wait()
            pltpu.sync_copy(bufs[cur], tx1_sh.at[cme.at[jj]], add=True)
            if jj == qc - 1 and q + 2 < nq:
                idescs[q + 2] = stage(q + 2)
        plsc.subcore_barrier()

        pltpu.sync_copy(
            tx1_sh.at[pl.ds(s * nps, nps), :],
            tx1_out.at[c, pl.ds(s * nps, nps), :],
        )

    return spmm


def _dense_kernel(dh, x_ref, tp_ref, nd_ref, w0_ref, w1_ref, bc_ref, o_ref):
    t = (tp_ref[0] + tp_ref[1]) * (-nd_ref[...])
    acc = jnp.dot(x_ref[...], w0_ref[...], preferred_element_type=jnp.float32)
    acc = acc + jnp.dot(t, w1_ref[...], preferred_element_type=jnp.float32)
    acc = acc + bc_ref[...]
    z = jax.nn.sigmoid(acc[:, :dh])
    h = jnp.tanh(acc[:, dh:])
    o_ref[...] = (1.0 - z) * h


def kernel(x, edge_index, W_xz, b_xz, W_hz, b_hz, W_xr, b_xr, W_hr, b_hr,
           W_xh, b_xh, W_hh, b_hh):
    n, d = x.shape
    dh = W_xz.shape[2]
    e = edge_index.shape[1]

    ept = _round_up(-(-e // NW), CH * 8)
    e_pad = ept * NW
    chunks_c = ept // CH
    chunks_a = e_pad // (NS * CH)
    n_pad = _round_up(n + 1, NW * 16)

    pad = e_pad - e
    sink = n + jnp.arange(pad, dtype=jnp.int32) % jnp.int32(n_pad - n)
    ei_p = jnp.concatenate(
        [edge_index, jnp.broadcast_to(sink, (2, pad))], axis=1
    ).reshape(2, -1, CH)
    x_p = jnp.zeros((n_pad, d), jnp.float32).at[:n].set(x)

    dis, xs = _make_stats_kernel(n_pad, d, chunks_a)(x_p, ei_p)
    tx1_parts = _make_spmm_kernel(n_pad, d, chunks_c)(xs, ei_p)

    w0c = jnp.concatenate([W_xz[0], W_xh[0]], axis=1)
    w1c = jnp.concatenate([W_xz[1], W_xh[1]], axis=1)
    bc = jnp.concatenate([b_xz + b_hz, b_xh + b_hh]).reshape(1, 2 * dh)

    bn = 1000
    grid = (n // bn,)
    out = pl.pallas_call(
        functools.partial(_dense_kernel, dh),
        grid=grid,
        in_specs=[
            pl.BlockSpec((bn, d), lambda i: (i, 0)),
            pl.BlockSpec((NC, bn, d), lambda i: (0, i, 0)),
            pl.BlockSpec((bn, 1), lambda i: (i, 0)),
            pl.BlockSpec((d, 2 * dh), lambda i: (0, 0)),
            pl.BlockSpec((d, 2 * dh), lambda i: (0, 0)),
            pl.BlockSpec((1, 2 * dh), lambda i: (0, 0)),
        ],
        out_specs=pl.BlockSpec((bn, dh), lambda i: (i, 0)),
        out_shape=jax.ShapeDtypeStruct((n, dh), jnp.float32),
    )(x, tx1_parts, dis.reshape(n_pad, 1), w0c, w1c, bc)
    return out


# speedup vs baseline: 1.0123x; 1.0123x over previous
import functools

import jax
import jax.numpy as jnp
from jax import lax
from jax.experimental import pallas as pl
from jax.experimental.pallas import tpu as pltpu
from jax.experimental.pallas import tpu_sc as plsc

NC = 2
NS = 16
NW = NC * NS
CH = 128


def _round_up(a, b):
    return (a + b - 1) // b * b


def _mesh():
    return plsc.VectorSubcoreMesh(
        core_axis_name="c", subcore_axis_name="s", num_cores=NC, num_subcores=NS
    )


def _make_stats_kernel(n_pad, d, chunks_a):
    npt = n_pad // NW
    nps = n_pad // NS

    @functools.partial(
        pl.kernel,
        out_type=[
            jax.ShapeDtypeStruct((n_pad,), jnp.float32),
            jax.ShapeDtypeStruct((n_pad, d), jnp.float32),
        ],
        mesh=_mesh(),
        scratch_types=[
            pltpu.VMEM_SHARED((n_pad,), jnp.float32),
            pltpu.VMEM((chunks_a, CH), jnp.int32),
            pltpu.VMEM((CH,), jnp.float32),
            pltpu.VMEM((nps,), jnp.float32),
            pltpu.VMEM((npt,), jnp.float32),
            pltpu.VMEM((npt,), jnp.float32),
            pltpu.VMEM((npt, d), jnp.float32),
            pltpu.SemaphoreType.DMA,
            pltpu.SemaphoreType.DMA,
        ],
    )
    def stats(x_hbm, ei_hbm, dis_out, xs_out,
              deg_sh, idx_v, ones_v, zz_v, deg_v, dis_v, xrow_v, sem, sem_x):
        c = lax.axis_index("c")
        s = lax.axis_index("s")
        w = c * NS + s

        xdesc = pltpu.async_copy(x_hbm.at[pl.ds(w * npt, npt), :], xrow_v, sem_x)

        def zloop(i, carry):
            zz_v[pl.ds(i * 16, 16)] = jnp.zeros((16,), jnp.float32)
            return carry

        lax.fori_loop(0, nps // 16, zloop, 0)
        pltpu.sync_copy(zz_v, deg_sh.at[pl.ds(s * nps, nps)])

        def oloop(i, carry):
            ones_v[pl.ds(i * 16, 16)] = jnp.ones((16,), jnp.float32)
            return carry

        lax.fori_loop(0, CH // 16, oloop, 0)
        pltpu.sync_copy(ei_hbm.at[0, pl.ds(s * chunks_a, chunks_a), :], idx_v)
        plsc.subcore_barrier()

        k = 160
        for g in range(0, chunks_a, k):
            descs = [
                pltpu.async_copy(ones_v, deg_sh.at[idx_v.at[j]], sem, add=True)
                for j in range(g, min(g + k, chunks_a))
            ]
            for dsc in descs:
                dsc.wait()
        plsc.subcore_barrier()

        pltpu.sync_copy(deg_sh.at[pl.ds(w * npt, npt)], deg_v)

        def dloop(i, carry):
            dvec = deg_v[pl.ds(i * 16, 16)]
            ib = lax.bitcast_convert_type(dvec, jnp.int32)
            y = lax.bitcast_convert_type(
                jnp.int32(0x5F3759DF) - (ib >> 1), jnp.float32
            )
            half = 0.5 * dvec
            y = y * (1.5 - half * y * y)
            y = y * (1.5 - half * y * y)
            y = y * (1.5 - half * y * y)
            dis_v[pl.ds(i * 16, 16)] = jnp.where(dvec > 0.5, y, 0.0)
            return carry

        lax.fori_loop(0, npt // 16, dloop, 0)
        pltpu.sync_copy(dis_v, dis_out.at[pl.ds(w * npt, npt)])

        xdesc.wait()

        def xloop(g, carry):
            dvec = dis_v[pl.ds(g * 16, 16)]
            for l in range(16):
                dv = dvec[l]
                r = g * 16 + l
                for k in range(d // 16):
                    xrow_v[r, pl.ds(k * 16, 16)] = (
                        xrow_v[r, pl.ds(k * 16, 16)] * dv
                    )
            return carry

        lax.fori_loop(0, npt // 16, xloop, 0)
        pltpu.sync_copy(xrow_v, xs_out.at[pl.ds(w * npt, npt), :])

    return stats


def _make_spmm_kernel(n_pad, d, chunks_c):
    nps = n_pad // NS

    nq = 10
    qc = chunks_c // nq

    @functools.partial(
        pl.kernel,
        out_type=jax.ShapeDtypeStruct((NC, n_pad, d), jnp.float32),
        mesh=_mesh(),
        scratch_types=[
            pltpu.VMEM_SHARED((n_pad, d), jnp.float32),
            pltpu.VMEM((qc, CH), jnp.int32),
            pltpu.VMEM((qc, CH), jnp.int32),
            pltpu.VMEM((qc, CH), jnp.int32),
            pltpu.VMEM((qc, CH), jnp.int32),
            pltpu.VMEM((CH, d), jnp.float32),
            pltpu.VMEM((CH, d), jnp.float32),
            pltpu.SemaphoreType.DMA,
            pltpu.SemaphoreType.DMA,
            pltpu.SemaphoreType.DMA,
            pltpu.SemaphoreType.DMA,
        ],
    )
    def spmm(xs_hbm, ei_hbm, tx1_out,
             tx1_sh, ridx0, cidx0, ridx1, cidx1, buf0, buf1,
             sem0, sem1, isem0, isem1):
        c = lax.axis_index("c")
        s = lax.axis_index("s")
        w = c * NS + s

        pairs = ((ridx0, cidx0, isem0), (ridx1, cidx1, isem1))

        def stage(q):
            r, ci, sm = pairs[q % 2]
            base = w * chunks_c + q * qc
            return (
                pltpu.async_copy(ei_hbm.at[0, pl.ds(base, qc), :], r, sm),
                pltpu.async_copy(ei_hbm.at[1, pl.ds(base, qc), :], ci, sm),
            )

        idescs = [None] * nq
        idescs[0] = stage(0)

        def zrow(r, carry):
            for k in range(d // 16):
                buf0[r, pl.ds(k * 16, 16)] = jnp.zeros((16,), jnp.float32)
            return carry

        lax.fori_loop(0, CH, zrow, 0)
        for i in range(nps // CH):
            pltpu.sync_copy(buf0, tx1_sh.at[pl.ds(s * nps + i * CH, CH), :])
        plsc.subcore_barrier()

        for dsc in idescs[0]:
            dsc.wait()
        if nq > 1:
            idescs[1] = stage(1)

        bufs = (buf0, buf1)
        sems = (sem0, sem1)
        g_descs = [None, None]
        g_descs[0] = pltpu.async_copy(xs_hbm.at[ridx0.at[0]], buf0, sem0)
        for j in range(chunks_c):
            q, jj = j // qc, j % qc
            cme = pairs[q % 2][1]
            cur, nxt = j % 2, (j + 1) % 2
            if j + 1 < chunks_c:
                q2 = (j + 1) // qc
                if q2 != q:
                    for dsc in idescs[q2]:
                        dsc.wait()
                rn = pairs[q2 % 2][0]
                g_descs[nxt] = pltpu.async_copy(
                    xs_hbm.at[rn.at[(j + 1) % qc]], bufs[nxt], sems[nxt]
                )
            g_descs[cur].wait()
            pltpu.sync_copy(bufs[cur], tx1_sh.at[cme.at[jj]], add=True)
            if jj == qc - 1 and q + 2 < nq:
                idescs[q + 2] = stage(q + 2)
        plsc.subcore_barrier()

        pltpu.sync_copy(
            tx1_sh.at[pl.ds(s * nps, nps), :],
            tx1_out.at[c, pl.ds(s * nps, nps), :],
        )

    return spmm


def _dense_kernel(dh, x_ref, tp_ref, nd_ref, w0_ref, w1_ref, bc_ref, o_ref):
    t = (tp_ref[0] + tp_ref[1]) * (-nd_ref[...])
    acc = jnp.dot(x_ref[...], w0_ref[...], preferred_element_type=jnp.float32)
    acc = acc + jnp.dot(t, w1_ref[...], preferred_element_type=jnp.float32)
    acc = acc + bc_ref[...]
    z = jax.nn.sigmoid(acc[:, :dh])
    h = jnp.tanh(acc[:, dh:])
    o_ref[...] = (1.0 - z) * h


def kernel(x, edge_index, W_xz, b_xz, W_hz, b_hz, W_xr, b_xr, W_hr, b_hr,
           W_xh, b_xh, W_hh, b_hh):
    n, d = x.shape
    dh = W_xz.shape[2]
    e = edge_index.shape[1]

    ept = _round_up(-(-e // NW), CH * 8)
    e_pad = ept * NW
    chunks_c = ept // CH
    chunks_a = e_pad // (NS * CH)
    n_pad = _round_up(n + 1, NW * 16)

    pad = e_pad - e
    sink = n + jnp.arange(pad, dtype=jnp.int32) % jnp.int32(n_pad - n)
    ei_p = jnp.concatenate(
        [edge_index, jnp.broadcast_to(sink, (2, pad))], axis=1
    ).reshape(2, -1, CH)
    x_p = jnp.zeros((n_pad, d), jnp.float32).at[:n].set(x)

    dis, xs = _make_stats_kernel(n_pad, d, chunks_a)(x_p, ei_p)
    tx1_parts = _make_spmm_kernel(n_pad, d, chunks_c)(xs, ei_p)

    w0c = jnp.concatenate([W_xz[0], W_xh[0]], axis=1)
    w1c = jnp.concatenate([W_xz[1], W_xh[1]], axis=1)
    bc = jnp.concatenate([b_xz + b_hz, b_xh + b_hh]).reshape(1, 2 * dh)

    bn = 2000
    grid = (n // bn,)
    out = pl.pallas_call(
        functools.partial(_dense_kernel, dh),
        grid=grid,
        in_specs=[
            pl.BlockSpec((bn, d), lambda i: (i, 0)),
            pl.BlockSpec((NC, bn, d), lambda i: (0, i, 0)),
            pl.BlockSpec((bn, 1), lambda i: (i, 0)),
            pl.BlockSpec((d, 2 * dh), lambda i: (0, 0)),
            pl.BlockSpec((d, 2 * dh), lambda i: (0, 0)),
            pl.BlockSpec((1, 2 * dh), lambda i: (0, 0)),
        ],
        out_specs=pl.BlockSpec((bn, dh), lambda i: (i, 0)),
        out_shape=jax.ShapeDtypeStruct((n, dh), jnp.float32),
    )(x, tx1_parts, dis.reshape(n_pad, 1), w0c, w1c, bc)
    return out


# speedup vs baseline: 1.0219x; 1.0095x over previous
import functools

import jax
import jax.numpy as jnp
from jax import lax
from jax.experimental import pallas as pl
from jax.experimental.pallas import tpu as pltpu
from jax.experimental.pallas import tpu_sc as plsc

NC = 2
NS = 16
NW = NC * NS
CH = 128


def _round_up(a, b):
    return (a + b - 1) // b * b


def _mesh():
    return plsc.VectorSubcoreMesh(
        core_axis_name="c", subcore_axis_name="s", num_cores=NC, num_subcores=NS
    )


def _make_stats_kernel(n_pad, d, chunks_a):
    npt = n_pad // NW
    nps = n_pad // NS

    @functools.partial(
        pl.kernel,
        out_type=[
            jax.ShapeDtypeStruct((n_pad,), jnp.float32),
            jax.ShapeDtypeStruct((n_pad, d), jnp.float32),
        ],
        mesh=_mesh(),
        scratch_types=[
            pltpu.VMEM_SHARED((n_pad,), jnp.float32),
            pltpu.VMEM((chunks_a, CH), jnp.int32),
            pltpu.VMEM((CH,), jnp.float32),
            pltpu.VMEM((nps,), jnp.float32),
            pltpu.VMEM((npt,), jnp.float32),
            pltpu.VMEM((npt,), jnp.float32),
            pltpu.VMEM((npt, d), jnp.float32),
            pltpu.SemaphoreType.DMA,
            pltpu.SemaphoreType.DMA,
        ],
    )
    def stats(x_hbm, ei_hbm, dis_out, xs_out,
              deg_sh, idx_v, ones_v, zz_v, deg_v, dis_v, xrow_v, sem, sem_x):
        c = lax.axis_index("c")
        s = lax.axis_index("s")
        w = c * NS + s

        xdesc = pltpu.async_copy(x_hbm.at[pl.ds(w * npt, npt), :], xrow_v, sem_x)

        def zloop(i, carry):
            zz_v[pl.ds(i * 16, 16)] = jnp.zeros((16,), jnp.float32)
            return carry

        lax.fori_loop(0, nps // 16, zloop, 0)
        pltpu.sync_copy(zz_v, deg_sh.at[pl.ds(s * nps, nps)])

        def oloop(i, carry):
            ones_v[pl.ds(i * 16, 16)] = jnp.ones((16,), jnp.float32)
            return carry

        lax.fori_loop(0, CH // 16, oloop, 0)
        pltpu.sync_copy(ei_hbm.at[0, pl.ds(s * chunks_a, chunks_a), :], idx_v)
        plsc.subcore_barrier()

        k = 40
        for g in range(0, chunks_a, k):
            descs = [
                pltpu.async_copy(ones_v, deg_sh.at[idx_v.at[j]], sem, add=True)
                for j in range(g, min(g + k, chunks_a))
            ]
            for dsc in descs:
                dsc.wait()
        plsc.subcore_barrier()

        pltpu.sync_copy(deg_sh.at[pl.ds(w * npt, npt)], deg_v)

        def dloop(i, carry):
            dvec = deg_v[pl.ds(i * 16, 16)]
            ib = lax.bitcast_convert_type(dvec, jnp.int32)
            y = lax.bitcast_convert_type(
                jnp.int32(0x5F3759DF) - (ib >> 1), jnp.float32
            )
            half = 0.5 * dvec
            y = y * (1.5 - half * y * y)
            y = y * (1.5 - half * y * y)
            y = y * (1.5 - half * y * y)
            dis_v[pl.ds(i * 16, 16)] = jnp.where(dvec > 0.5, y, 0.0)
            return carry

        lax.fori_loop(0, npt // 16, dloop, 0)
        pltpu.sync_copy(dis_v, dis_out.at[pl.ds(w * npt, npt)])

        xdesc.wait()

        def xloop(g, carry):
            dvec = dis_v[pl.ds(g * 16, 16)]
            for l in range(16):
                dv = dvec[l]
                r = g * 16 + l
                for k in range(d // 16):
                    xrow_v[r, pl.ds(k * 16, 16)] = (
                        xrow_v[r, pl.ds(k * 16, 16)] * dv
                    )
            return carry

        lax.fori_loop(0, npt // 16, xloop, 0)
        pltpu.sync_copy(xrow_v, xs_out.at[pl.ds(w * npt, npt), :])

    return stats


def _make_spmm_kernel(n_pad, d, chunks_c):
    nps = n_pad // NS

    nq = 10
    qc = chunks_c // nq

    @functools.partial(
        pl.kernel,
        out_type=jax.ShapeDtypeStruct((NC, n_pad, d), jnp.float32),
        mesh=_mesh(),
        scratch_types=[
            pltpu.VMEM_SHARED((n_pad, d), jnp.float32),
            pltpu.VMEM((qc, CH), jnp.int32),
            pltpu.VMEM((qc, CH), jnp.int32),
            pltpu.VMEM((qc, CH), jnp.int32),
            pltpu.VMEM((qc, CH), jnp.int32),
            pltpu.VMEM((CH, d), jnp.float32),
            pltpu.VMEM((CH, d), jnp.float32),
            pltpu.SemaphoreType.DMA,
            pltpu.SemaphoreType.DMA,
            pltpu.SemaphoreType.DMA,
            pltpu.SemaphoreType.DMA,
        ],
    )
    def spmm(xs_hbm, ei_hbm, tx1_out,
             tx1_sh, ridx0, cidx0, ridx1, cidx1, buf0, buf1,
             sem0, sem1, isem0, isem1):
        c = lax.axis_index("c")
        s = lax.axis_index("s")
        w = c * NS + s

        pairs = ((ridx0, cidx0, isem0), (ridx1, cidx1, isem1))

        def stage(q):
            r, ci, sm = pairs[q % 2]
            base = w * chunks_c + q * qc
            return (
                pltpu.async_copy(ei_hbm.at[0, pl.ds(base, qc), :], r, sm),
                pltpu.async_copy(ei_hbm.at[1, pl.ds(base, qc), :], ci, sm),
            )

        idescs = [None] * nq
        idescs[0] = stage(0)

        def zrow(r, carry):
            for k in range(d // 16):
                buf0[r, pl.ds(k * 16, 16)] = jnp.zeros((16,), jnp.float32)
            return carry

        lax.fori_loop(0, CH, zrow, 0)
        for i in range(nps // CH):
            pltpu.sync_copy(buf0, tx1_sh.at[pl.ds(s * nps + i * CH, CH), :])
        plsc.subcore_barrier()

        for dsc in idescs[0]:
            dsc.wait()
        if nq > 1:
            idescs[1] = stage(1)

        bufs = (buf0, buf1)
        sems = (sem0, sem1)
        g_descs = [None, None]
        g_descs[0] = pltpu.async_copy(xs_hbm.at[ridx0.at[0]], buf0, sem0)
        for j in range(chunks_c):
            q, jj = j // qc, j % qc
            cme = pairs[q % 2][1]
            cur, nxt = j % 2, (j + 1) % 2
            if j + 1 < chunks_c:
                q2 = (j + 1) // qc
                if q2 != q:
                    for dsc in idescs[q2]:
                        dsc.wait()
                rn = pairs[q2 % 2][0]
                g_descs[nxt] = pltpu.async_copy(
                    xs_hbm.at[rn.at[(j + 1) % qc]], bufs[nxt], sems[nxt]
                )
            g_descs[cur].wait()
            pltpu.sync_copy(bufs[cur], tx1_sh.at[cme.at[jj]], add=True)
            if jj == qc - 1 and q + 2 < nq:
                idescs[q + 2] = stage(q + 2)
        plsc.subcore_barrier()

        pltpu.sync_copy(
            tx1_sh.at[pl.ds(s * nps, nps), :],
            tx1_out.at[c, pl.ds(s * nps, nps), :],
        )

    return spmm


def _dense_kernel(dh, x_ref, tp_ref, nd_ref, w0_ref, w1_ref, bc_ref, o_ref):
    t = (tp_ref[0] + tp_ref[1]) * (-nd_ref[...])
    acc = jnp.dot(x_ref[...], w0_ref[...], preferred_element_type=jnp.float32)
    acc = acc + jnp.dot(t, w1_ref[...], preferred_element_type=jnp.float32)
    acc = acc + bc_ref[...]
    z = jax.nn.sigmoid(acc[:, :dh])
    h = jnp.tanh(acc[:, dh:])
    o_ref[...] = (1.0 - z) * h


def kernel(x, edge_index, W_xz, b_xz, W_hz, b_hz, W_xr, b_xr, W_hr, b_hr,
           W_xh, b_xh, W_hh, b_hh):
    n, d = x.shape
    dh = W_xz.shape[2]
    e = edge_index.shape[1]

    ept = _round_up(-(-e // NW), CH * 8)
    e_pad = ept * NW
    chunks_c = ept // CH
    chunks_a = e_pad // (NS * CH)
    n_pad = _round_up(n + 1, NW * 16)

    pad = e_pad - e
    sink = n + jnp.arange(pad, dtype=jnp.int32) % jnp.int32(n_pad - n)
    ei_p = jnp.concatenate(
        [edge_index, jnp.broadcast_to(sink, (2, pad))], axis=1
    ).reshape(2, -1, CH)
    x_p = jnp.zeros((n_pad, d), jnp.float32).at[:n].set(x)

    dis, xs = _make_stats_kernel(n_pad, d, chunks_a)(x_p, ei_p)
    tx1_parts = _make_spmm_kernel(n_pad, d, chunks_c)(xs, ei_p)

    w0c = jnp.concatenate([W_xz[0], W_xh[0]], axis=1)
    w1c = jnp.concatenate([W_xz[1], W_xh[1]], axis=1)
    bc = jnp.concatenate([b_xz + b_hz, b_xh + b_hh]).reshape(1, 2 * dh)

    bn = 5000
    grid = (n // bn,)
    out = pl.pallas_call(
        functools.partial(_dense_kernel, dh),
        grid=grid,
        in_specs=[
            pl.BlockSpec((bn, d), lambda i: (i, 0)),
            pl.BlockSpec((NC, bn, d), lambda i: (0, i, 0)),
            pl.BlockSpec((bn, 1), lambda i: (i, 0)),
            pl.BlockSpec((d, 2 * dh), lambda i: (0, 0)),
            pl.BlockSpec((d, 2 * dh), lambda i: (0, 0)),
            pl.BlockSpec((1, 2 * dh), lambda i: (0, 0)),
        ],
        out_specs=pl.BlockSpec((bn, dh), lambda i: (i, 0)),
        out_shape=jax.ShapeDtypeStruct((n, dh), jnp.float32),
    )(x, tx1_parts, dis.reshape(n_pad, 1), w0c, w1c, bc)
    return out
